# baseline (speedup 1.0000x reference)
import functools

import jax
import jax.numpy as jnp
from jax.experimental import pallas as pl
from jax.experimental.pallas import tpu as pltpu
from jax.experimental.pallas import tpu_sc as plsc

K_BLOCK = 512
SC_WINDOW = 128
SC_LANES = 16


def _argmin_dist_kernel(xm2_ref, cb_ref, csq_ref, xsq_ref, idx_ref,
                        rm_ref, ri_ref, *, n_steps):
    k = pl.program_id(0)

    @pl.when(k == 0)
    def _init():
        rm_ref[...] = jnp.full(rm_ref.shape, jnp.inf, dtype=rm_ref.dtype)

    cross2 = jax.lax.dot_general(
        xm2_ref[...], cb_ref[...],
        dimension_numbers=(((1,), (1,)), ((), ())),
        preferred_element_type=jnp.float32,
    )
    dist = (xsq_ref[...] + cross2) + csq_ref[0]

    iota = jax.lax.broadcasted_iota(jnp.int32, dist.shape, 1) + k * K_BLOCK
    mask = dist < rm_ref[...]
    rm_ref[...] = jnp.where(mask, dist, rm_ref[...])
    ri_ref[...] = jnp.where(mask, iota, ri_ref[...])

    @pl.when(k == n_steps - 1)
    def _finalize():
        rm = rm_ref[...]
        ri = ri_ref[...]
        rowmin = jnp.min(rm, axis=1, keepdims=True)
        cand = jnp.where(rm == rowmin, ri, jnp.int32(2**30))
        idx_ref[...] = jnp.min(cand, axis=1, keepdims=True)


def _tc_argmin(xm2, codebook, c_sq3, x_sq):
    batch, _ = xm2.shape
    n_codes = codebook.shape[0]
    n_steps = n_codes // K_BLOCK
    grid_kernel = functools.partial(_argmin_dist_kernel, n_steps=n_steps)
    return pl.pallas_call(
        grid_kernel,
        grid=(n_steps,),
        in_specs=[
            pl.BlockSpec((batch, xm2.shape[1]), lambda k: (0, 0)),
            pl.BlockSpec((K_BLOCK, codebook.shape[1]), lambda k: (k, 0)),
            pl.BlockSpec((1, 1, K_BLOCK), lambda k: (k, 0, 0)),
            pl.BlockSpec((batch, 1), lambda k: (0, 0)),
        ],
        out_specs=pl.BlockSpec((batch, 1), lambda k: (0, 0)),
        out_shape=jax.ShapeDtypeStruct((batch, 1), jnp.int32),
        scratch_shapes=[
            pltpu.VMEM((batch, K_BLOCK), jnp.float32),
            pltpu.VMEM((batch, K_BLOCK), jnp.int32),
        ],
        compiler_params=pltpu.CompilerParams(
            dimension_semantics=("arbitrary",),
        ),
    )(xm2, codebook, c_sq3, x_sq)


def _sc_gather_sub(codebook, idx2d, prev_residual):
    batch, dim = prev_residual.shape
    half = dim // 2
    cb_h = codebook.reshape(codebook.shape[0] * 2, half)
    x_h = prev_residual.reshape(batch * 2, half)
    idx = idx2d.reshape(batch)
    idx_h = (idx[:, None] * 2 + jnp.arange(2, dtype=idx.dtype)[None, :]
             ).reshape(1, batch * 2)
    mesh = plsc.VectorSubcoreMesh(core_axis_name="core",
                                  subcore_axis_name="subcore")

    @functools.partial(
        pl.kernel,
        out_type=jax.ShapeDtypeStruct((batch * 2, half), jnp.float32),
        mesh=mesh,
    )
    def sc_kernel(cb_hbm, i_hbm, x_hbm, o_hbm):
        def body(i_vmem, x_vmem, o_vmem):
            pltpu.sync_copy(cb_hbm.at[i_vmem.at[0]], o_vmem)

            @pl.loop(0, SC_WINDOW)
            def _(r):
                @pl.loop(0, half, step=SC_LANES)
                def _(c):
                    slc = (pl.ds(r, 1), pl.ds(c, SC_LANES))
                    o_vmem.at[*slc][...] = (
                        x_vmem.at[*slc][...] - o_vmem.at[*slc][...]
                    )

        pltpu.emit_pipeline(
            body,
            grid=(batch * 2 // SC_WINDOW,),
            in_specs=[
                pl.BlockSpec((1, SC_WINDOW), lambda i: (0, i)),
                pl.BlockSpec((SC_WINDOW, half), lambda i: (i, 0)),
            ],
            out_specs=[pl.BlockSpec((SC_WINDOW, half), lambda i: (i, 0))],
            core_axis_name=("core", "subcore"),
            dimension_semantics=(pltpu.PARALLEL,),
        )(i_hbm, x_hbm, o_hbm)

    return sc_kernel(cb_h, idx_h, x_h).reshape(batch, dim)


def kernel(previous_residual, codebook_embeddings):
    batch = previous_residual.shape[0]
    n_codes = codebook_embeddings.shape[0]

    x_sq = jnp.sum(jnp.square(previous_residual), axis=-1, keepdims=True)
    c_sq = jnp.sum(jnp.square(codebook_embeddings), axis=-1)
    c_sq3 = c_sq.reshape(n_codes // K_BLOCK, 1, K_BLOCK)
    xm2 = previous_residual * (-2.0)

    idx2d = _tc_argmin(xm2, codebook_embeddings, c_sq3, x_sq)
    idx = idx2d.reshape(batch)

    next_residual = _sc_gather_sub(
        codebook_embeddings, idx2d.reshape(1, batch), previous_residual)
    return (idx, next_residual)


# speedup vs baseline: 1.4944x; 1.4944x over previous
import functools

import jax
import jax.numpy as jnp
from jax.experimental import pallas as pl
from jax.experimental.pallas import tpu as pltpu
from jax.experimental.pallas import tpu_sc as plsc

K_BLOCK = 512
SC_WINDOW = 128


def _argmin_dist_kernel(x_ref, cb_ref, idx_ref, rm_ref, ri_ref,
                        xm2_ref, xsq_ref, *, n_steps):
    k = pl.program_id(0)

    @pl.when(k == 0)
    def _init():
        rm_ref[...] = jnp.full(rm_ref.shape, jnp.inf, dtype=rm_ref.dtype)
        x = x_ref[...]
        xm2_ref[...] = x * (-2.0)
        xsq_ref[...] = jnp.sum(jnp.square(x), axis=1, keepdims=True)

    cb = cb_ref[...]
    c_sq = jnp.sum(jnp.square(cb), axis=1, keepdims=True)
    c_sq_row = jax.lax.transpose(c_sq, (1, 0))

    cross2 = jax.lax.dot_general(
        xm2_ref[...], cb,
        dimension_numbers=(((1,), (1,)), ((), ())),
        preferred_element_type=jnp.float32,
    )
    dist = (xsq_ref[...] + cross2) + c_sq_row

    iota = jax.lax.broadcasted_iota(jnp.int32, dist.shape, 1) + k * K_BLOCK
    mask = dist < rm_ref[...]
    rm_ref[...] = jnp.where(mask, dist, rm_ref[...])
    ri_ref[...] = jnp.where(mask, iota, ri_ref[...])

    @pl.when(k == n_steps - 1)
    def _finalize():
        rm = rm_ref[...]
        ri = ri_ref[...]
        rowmin = jnp.min(rm, axis=1, keepdims=True)
        cand = jnp.where(rm == rowmin, ri, jnp.int32(2**30))
        idx_col = jnp.min(cand, axis=1, keepdims=True)
        idx_ref[...] = jax.lax.transpose(idx_col, (1, 0))


def _tc_argmin(x, codebook):
    batch, dim = x.shape
    n_codes = codebook.shape[0]
    n_steps = n_codes // K_BLOCK
    grid_kernel = functools.partial(_argmin_dist_kernel, n_steps=n_steps)
    return pl.pallas_call(
        grid_kernel,
        grid=(n_steps,),
        in_specs=[
            pl.BlockSpec((batch, dim), lambda k: (0, 0)),
            pl.BlockSpec((K_BLOCK, dim), lambda k: (k, 0)),
        ],
        out_specs=pl.BlockSpec((1, batch), lambda k: (0, 0)),
        out_shape=jax.ShapeDtypeStruct((1, batch), jnp.int32),
        scratch_shapes=[
            pltpu.VMEM((batch, K_BLOCK), jnp.float32),
            pltpu.VMEM((batch, K_BLOCK), jnp.int32),
            pltpu.VMEM((batch, dim), jnp.float32),
            pltpu.VMEM((batch, 1), jnp.float32),
        ],
        compiler_params=pltpu.CompilerParams(
            dimension_semantics=("arbitrary",),
        ),
    )(x, codebook)


def _sc_gather(codebook, idx_row):
    n_codes, dim = codebook.shape
    batch = idx_row.shape[1]
    mesh = plsc.VectorSubcoreMesh(core_axis_name="core",
                                  subcore_axis_name="subcore")

    @functools.partial(
        pl.kernel,
        out_type=jax.ShapeDtypeStruct((batch, dim), jnp.float32),
        mesh=mesh,
    )
    def sc_kernel(cb_hbm, i_hbm, o_hbm):
        def body(i_vmem, o_vmem):
            pltpu.sync_copy(cb_hbm.at[i_vmem.at[0]], o_vmem)

        pltpu.emit_pipeline(
            body,
            grid=(batch // SC_WINDOW,),
            in_specs=[
                pl.BlockSpec((1, SC_WINDOW), lambda i: (0, i)),
            ],
            out_specs=[pl.BlockSpec((SC_WINDOW, dim), lambda i: (i, 0))],
            core_axis_name=("core", "subcore"),
            dimension_semantics=(pltpu.PARALLEL,),
        )(i_hbm, o_hbm)

    return sc_kernel(codebook, idx_row)


def kernel(previous_residual, codebook_embeddings):
    batch = previous_residual.shape[0]
    idx_row = _tc_argmin(previous_residual, codebook_embeddings)
    gathered = _sc_gather(codebook_embeddings, idx_row)
    next_residual = previous_residual - gathered
    return (idx_row.reshape(batch), next_residual)
